# Initial kernel scaffold; baseline (speedup 1.0000x reference)
import jax
import jax.numpy as jnp
from jax.experimental import pallas as pl


def kernel(x, y, idx1, sign1, idx2, sign2, outConn):
    raise NotImplementedError("write your pallas kernel here")


# speedup vs baseline: 7.3058x; 7.3058x over previous
import functools

import jax
import jax.numpy as jnp
from jax import lax
from jax.experimental import pallas as pl
from jax.experimental.pallas import tpu as pltpu
from jax.experimental.pallas import tpu_sc as plsc

NUM_BITS = 16
THETA1 = 4.0
THETA2 = 4.0

_NC = 2
_NS = 16
_NW = _NC * _NS


def _sc_densify(idx_flat, sign_flat, n_rows, n_cols, k_syn, chunk_rows):
    rows_per_w = n_rows // _NW
    n_chunks = rows_per_w // chunk_rows
    n_groups = k_syn // 16
    mesh = plsc.VectorSubcoreMesh(core_axis_name="c", subcore_axis_name="s")

    @functools.partial(
        pl.kernel,
        out_type=jax.ShapeDtypeStruct((n_rows * n_cols,), jnp.float32),
        mesh=mesh,
        compiler_params=pltpu.CompilerParams(needs_layout_passes=False),
        scratch_types=[
            pltpu.VMEM((chunk_rows * n_cols,), jnp.float32),
            pltpu.VMEM((rows_per_w * k_syn,), jnp.int32),
            pltpu.VMEM((rows_per_w * k_syn,), jnp.float32),
        ],
    )
    def dens(idx_hbm, sign_hbm, w_hbm, buf, idxv, sgnv):
        wid = lax.axis_index("s") * _NC + lax.axis_index("c")
        base_syn = wid * rows_per_w * k_syn
        pltpu.sync_copy(idx_hbm.at[pl.ds(base_syn, rows_per_w * k_syn)], idxv)
        pltpu.sync_copy(sign_hbm.at[pl.ds(base_syn, rows_per_w * k_syn)], sgnv)
        zeros16 = jnp.zeros((16,), jnp.float32)

        def zero_body(i, carry):
            buf[pl.ds(i * 16, 16)] = zeros16
            return carry

        lax.fori_loop(0, chunk_rows * n_cols // 16, zero_body, 0)

        def chunk_body(c, carry):
            row0 = c * chunk_rows
            for r in range(chunk_rows):
                for g in range(n_groups):
                    s = (row0 + r) * k_syn + g * 16
                    iv = idxv[pl.ds(s, 16)]
                    sv = sgnv[pl.ds(s, 16)]
                    plsc.addupdate_scatter(buf, [iv + r * n_cols], sv)
            hbm_base = (wid * rows_per_w + row0) * n_cols
            pltpu.sync_copy(buf, w_hbm.at[pl.ds(hbm_base, chunk_rows * n_cols)])
            for r in range(chunk_rows):
                for g in range(n_groups):
                    s = (row0 + r) * k_syn + g * 16
                    iv = idxv[pl.ds(s, 16)]
                    plsc.store_scatter(buf, [iv + r * n_cols], zeros16)
            return carry

        lax.fori_loop(0, n_chunks, chunk_body, 0)

    return dens(idx_flat, sign_flat)


def _tc_encode(x_t_pad, thr_col):
    fp, b = x_t_pad.shape
    ep = fp * NUM_BITS

    def body(x_ref, t_ref, o_ref):
        xp = x_ref[...]
        xe = jnp.broadcast_to(xp[:, None, :], (fp, NUM_BITS, b)).reshape(ep, b)
        o_ref[...] = (xe > t_ref[...]).astype(jnp.bfloat16)

    return pl.pallas_call(
        body,
        out_shape=jax.ShapeDtypeStruct((ep, b), jnp.bfloat16),
    )(x_t_pad, thr_col)


def _tc_layer(w, act, theta, block_h):
    h, d = w.shape
    b = act.shape[1]

    def body(w_ref, a_ref, o_ref):
        wb = w_ref[...].astype(jnp.bfloat16)
        z = jnp.dot(wb, a_ref[...], preferred_element_type=jnp.float32)
        o_ref[...] = (z >= theta).astype(jnp.bfloat16)

    return pl.pallas_call(
        body,
        grid=(h // block_h,),
        in_specs=[
            pl.BlockSpec((block_h, d), lambda i: (i, 0)),
            pl.BlockSpec((d, b), lambda i: (0, 0)),
        ],
        out_specs=pl.BlockSpec((block_h, b), lambda i: (i, 0)),
        out_shape=jax.ShapeDtypeStruct((h, b), jnp.bfloat16),
    )(w, act)


def _tc_upd(a1, a2, y_col, c_pad, block_h):
    h, b = a1.shape

    def body(a1_ref, a2_ref, y_ref, u1_ref, u2_ref):
        cls = lax.broadcasted_iota(jnp.int32, (b, c_pad), 1)
        onehot = (y_ref[...] == cls).astype(jnp.bfloat16)
        u1_ref[...] = jnp.dot(a1_ref[...], onehot, preferred_element_type=jnp.float32)
        u2_ref[...] = jnp.dot(a2_ref[...], onehot, preferred_element_type=jnp.float32)

    return pl.pallas_call(
        body,
        grid=(h // block_h,),
        in_specs=[
            pl.BlockSpec((block_h, b), lambda i: (i, 0)),
            pl.BlockSpec((block_h, b), lambda i: (i, 0)),
            pl.BlockSpec((b, 1), lambda i: (0, 0)),
        ],
        out_specs=[
            pl.BlockSpec((block_h, c_pad), lambda i: (i, 0)),
            pl.BlockSpec((block_h, c_pad), lambda i: (i, 0)),
        ],
        out_shape=[
            jax.ShapeDtypeStruct((h, c_pad), jnp.float32),
            jax.ShapeDtypeStruct((h, c_pad), jnp.float32),
        ],
    )(a1, a2, y_col)


def _tc_scores(oc0t, oc1t, u1t, u2t, a1, a2, block_h):
    c_pad, h = oc0t.shape
    b = a1.shape[1]

    def body(o0_ref, o1_ref, t1_ref, t2_ref, a1_ref, a2_ref, s_ref):
        @pl.when(pl.program_id(0) == 0)
        def _():
            s_ref[...] = jnp.zeros_like(s_ref)

        m0 = o0_ref[...] + t1_ref[...]
        m1 = o1_ref[...] + t2_ref[...]
        s_ref[...] += jnp.dot(m0, a1_ref[...].astype(jnp.float32),
                              preferred_element_type=jnp.float32)
        s_ref[...] += jnp.dot(m1, a2_ref[...].astype(jnp.float32),
                              preferred_element_type=jnp.float32)

    return pl.pallas_call(
        body,
        grid=(h // block_h,),
        in_specs=[
            pl.BlockSpec((c_pad, block_h), lambda i: (0, i)),
            pl.BlockSpec((c_pad, block_h), lambda i: (0, i)),
            pl.BlockSpec((c_pad, block_h), lambda i: (0, i)),
            pl.BlockSpec((c_pad, block_h), lambda i: (0, i)),
            pl.BlockSpec((block_h, b), lambda i: (i, 0)),
            pl.BlockSpec((block_h, b), lambda i: (i, 0)),
        ],
        out_specs=pl.BlockSpec((c_pad, b), lambda i: (0, 0)),
        out_shape=jax.ShapeDtypeStruct((c_pad, b), jnp.float32),
    )(oc0t, oc1t, u1t, u2t, a1, a2)


def kernel(x, y, idx1, sign1, idx2, sign2, outConn):
    b, f = x.shape
    h, k_syn = idx1.shape
    c = outConn.shape[-1]
    e = f * NUM_BITS

    f_pad = ((f + 7) // 8) * 8
    e_pad = f_pad * NUM_BITS
    c_pad = 16

    x_t = jnp.pad(x.T, ((0, f_pad - f), (0, 0)))
    thr = jnp.linspace(0.0, 1.0, NUM_BITS, dtype=jnp.float32)
    thr_col = jnp.pad(jnp.tile(thr, f), (0, e_pad - e),
                      constant_values=2.0).reshape(e_pad, 1)
    y_col = y.reshape(b, 1)
    oc_t = jnp.pad(outConn.transpose(0, 2, 1), ((0, 0), (0, c_pad - c), (0, 0)))

    w1t = _sc_densify(idx1.reshape(-1), sign1.reshape(-1),
                      h, e_pad, k_syn, 16).reshape(h, e_pad)
    w2t = _sc_densify(idx2.reshape(-1), sign2.reshape(-1),
                      h, h, k_syn, 8).reshape(h, h)

    enc_t = _tc_encode(x_t, thr_col)
    a1 = _tc_layer(w1t, enc_t, THETA1, 256)
    a2 = _tc_layer(w2t, a1, THETA2, 256)
    u1, u2 = _tc_upd(a1, a2, y_col, c_pad, 256)
    s_t = _tc_scores(oc_t[0], oc_t[1], u1.T, u2.T, a1, a2, 256)
    return s_t[:c, :].T


# speedup vs baseline: 11.3611x; 1.5551x over previous
import functools

import jax
import jax.numpy as jnp
from jax import lax
from jax.experimental import pallas as pl
from jax.experimental.pallas import tpu as pltpu
from jax.experimental.pallas import tpu_sc as plsc

NUM_BITS = 16
THETA1 = 4.0
THETA2 = 4.0

_NC = 2
_NS = 16
_NW = _NC * _NS


def _sc_densify(idx_flat, sign_flat, n_rows, n_cols, k_syn, chunk_rows):
    rows_per_w = n_rows // _NW
    n_chunks = rows_per_w // chunk_rows
    n_groups = k_syn // 16
    mesh = plsc.VectorSubcoreMesh(core_axis_name="c", subcore_axis_name="s")

    @functools.partial(
        pl.kernel,
        out_type=jax.ShapeDtypeStruct((n_rows, n_cols), jnp.float32),
        mesh=mesh,
        compiler_params=pltpu.CompilerParams(
            needs_layout_passes=False, use_tc_tiling_on_sc=True),
        scratch_types=[
            pltpu.VMEM((chunk_rows, n_cols), jnp.float32),
            pltpu.VMEM((rows_per_w * k_syn,), jnp.int32),
            pltpu.VMEM((rows_per_w * k_syn,), jnp.float32),
        ],
    )
    def dens(idx_hbm, sign_hbm, w_hbm, buf, idxv, sgnv):
        wid = lax.axis_index("s") * _NC + lax.axis_index("c")
        base_syn = wid * rows_per_w * k_syn
        pltpu.sync_copy(idx_hbm.at[pl.ds(base_syn, rows_per_w * k_syn)], idxv)
        pltpu.sync_copy(sign_hbm.at[pl.ds(base_syn, rows_per_w * k_syn)], sgnv)
        zeros16 = jnp.zeros((16,), jnp.float32)

        def zero_body(i, carry):
            r = i // (n_cols // 16)
            j = i % (n_cols // 16)
            buf[r, pl.ds(j * 16, 16)] = zeros16
            return carry

        lax.fori_loop(0, chunk_rows * n_cols // 16, zero_body, 0)

        def chunk_body(c, carry):
            row0 = c * chunk_rows
            for r in range(chunk_rows):
                rv = jnp.full((16,), r, jnp.int32)
                for g in range(n_groups):
                    s = (row0 + r) * k_syn + g * 16
                    iv = idxv[pl.ds(s, 16)]
                    sv = sgnv[pl.ds(s, 16)]
                    plsc.addupdate_scatter(buf, [rv, iv], sv)
            pltpu.sync_copy(
                buf, w_hbm.at[pl.ds(wid * rows_per_w + row0, chunk_rows)])
            for r in range(chunk_rows):
                rv = jnp.full((16,), r, jnp.int32)
                for g in range(n_groups):
                    s = (row0 + r) * k_syn + g * 16
                    iv = idxv[pl.ds(s, 16)]
                    plsc.store_scatter(buf, [rv, iv], zeros16)
            return carry

        lax.fori_loop(0, n_chunks, chunk_body, 0)

    return dens(idx_flat, sign_flat)


def _tc_encode(x_t_pad, thr_col):
    fp, b = x_t_pad.shape
    ep = fp * NUM_BITS

    def body(x_ref, t_ref, o_ref):
        xp = x_ref[...]
        xe = jnp.broadcast_to(xp[:, None, :], (fp, NUM_BITS, b)).reshape(ep, b)
        o_ref[...] = (xe > t_ref[...]).astype(jnp.bfloat16)

    return pl.pallas_call(
        body,
        out_shape=jax.ShapeDtypeStruct((ep, b), jnp.bfloat16),
    )(x_t_pad, thr_col)


def _tc_layer(w, act, theta, block_h):
    h, d = w.shape
    b = act.shape[1]

    def body(w_ref, a_ref, o_ref):
        wb = w_ref[...].astype(jnp.bfloat16)
        z = jnp.dot(wb, a_ref[...], preferred_element_type=jnp.float32)
        o_ref[...] = (z >= theta).astype(jnp.bfloat16)

    return pl.pallas_call(
        body,
        grid=(h // block_h,),
        in_specs=[
            pl.BlockSpec((block_h, d), lambda i: (i, 0)),
            pl.BlockSpec((d, b), lambda i: (0, 0)),
        ],
        out_specs=pl.BlockSpec((block_h, b), lambda i: (i, 0)),
        out_shape=jax.ShapeDtypeStruct((h, b), jnp.bfloat16),
    )(w, act)


def _tc_upd(a1, a2, y_col, c_pad, block_h):
    h, b = a1.shape

    def body(a1_ref, a2_ref, y_ref, u1_ref, u2_ref):
        cls = lax.broadcasted_iota(jnp.int32, (b, c_pad), 1)
        onehot = (y_ref[...] == cls).astype(jnp.bfloat16)
        u1_ref[...] = jnp.dot(a1_ref[...], onehot, preferred_element_type=jnp.float32)
        u2_ref[...] = jnp.dot(a2_ref[...], onehot, preferred_element_type=jnp.float32)

    return pl.pallas_call(
        body,
        grid=(h // block_h,),
        in_specs=[
            pl.BlockSpec((block_h, b), lambda i: (i, 0)),
            pl.BlockSpec((block_h, b), lambda i: (i, 0)),
            pl.BlockSpec((b, 1), lambda i: (0, 0)),
        ],
        out_specs=[
            pl.BlockSpec((block_h, c_pad), lambda i: (i, 0)),
            pl.BlockSpec((block_h, c_pad), lambda i: (i, 0)),
        ],
        out_shape=[
            jax.ShapeDtypeStruct((h, c_pad), jnp.float32),
            jax.ShapeDtypeStruct((h, c_pad), jnp.float32),
        ],
    )(a1, a2, y_col)


def _tc_scores(oc0t, oc1t, u1t, u2t, a1, a2, block_h):
    c_pad, h = oc0t.shape
    b = a1.shape[1]

    def body(o0_ref, o1_ref, t1_ref, t2_ref, a1_ref, a2_ref, s_ref):
        @pl.when(pl.program_id(0) == 0)
        def _():
            s_ref[...] = jnp.zeros_like(s_ref)

        m0 = o0_ref[...] + t1_ref[...]
        m1 = o1_ref[...] + t2_ref[...]
        s_ref[...] += jnp.dot(m0, a1_ref[...].astype(jnp.float32),
                              preferred_element_type=jnp.float32)
        s_ref[...] += jnp.dot(m1, a2_ref[...].astype(jnp.float32),
                              preferred_element_type=jnp.float32)

    return pl.pallas_call(
        body,
        grid=(h // block_h,),
        in_specs=[
            pl.BlockSpec((c_pad, block_h), lambda i: (0, i)),
            pl.BlockSpec((c_pad, block_h), lambda i: (0, i)),
            pl.BlockSpec((c_pad, block_h), lambda i: (0, i)),
            pl.BlockSpec((c_pad, block_h), lambda i: (0, i)),
            pl.BlockSpec((block_h, b), lambda i: (i, 0)),
            pl.BlockSpec((block_h, b), lambda i: (i, 0)),
        ],
        out_specs=pl.BlockSpec((c_pad, b), lambda i: (0, 0)),
        out_shape=jax.ShapeDtypeStruct((c_pad, b), jnp.float32),
    )(oc0t, oc1t, u1t, u2t, a1, a2)


def kernel(x, y, idx1, sign1, idx2, sign2, outConn):
    b, f = x.shape
    h, k_syn = idx1.shape
    c = outConn.shape[-1]
    e = f * NUM_BITS

    f_pad = ((f + 7) // 8) * 8
    e_pad = f_pad * NUM_BITS
    c_pad = 16

    x_t = jnp.pad(x.T, ((0, f_pad - f), (0, 0)))
    thr = jnp.linspace(0.0, 1.0, NUM_BITS, dtype=jnp.float32)
    thr_col = jnp.pad(jnp.tile(thr, f), (0, e_pad - e),
                      constant_values=2.0).reshape(e_pad, 1)
    y_col = y.reshape(b, 1)
    oc_t = jnp.pad(outConn.transpose(0, 2, 1), ((0, 0), (0, c_pad - c), (0, 0)))

    w1t = _sc_densify(idx1.reshape(-1), sign1.reshape(-1),
                      h, e_pad, k_syn, 16)
    w2t = _sc_densify(idx2.reshape(-1), sign2.reshape(-1),
                      h, h, k_syn, 8)

    enc_t = _tc_encode(x_t, thr_col)
    a1 = _tc_layer(w1t, enc_t, THETA1, 256)
    a2 = _tc_layer(w2t, a1, THETA2, 256)
    u1, u2 = _tc_upd(a1, a2, y_col, c_pad, 256)
    s_t = _tc_scores(oc_t[0], oc_t[1], u1.T, u2.T, a1, a2, 256)
    return s_t[:c, :].T


# speedup vs baseline: 12.6554x; 1.1139x over previous
import functools

import jax
import jax.numpy as jnp
from jax import lax
from jax.experimental import pallas as pl
from jax.experimental.pallas import tpu as pltpu
from jax.experimental.pallas import tpu_sc as plsc

NUM_BITS = 16
THETA1 = 4.0
THETA2 = 4.0

_NC = 2
_NS = 16
_NW = _NC * _NS


def _sc_densify(idx_flat, sign_flat, n_rows, n_cols, k_syn, chunk_rows):
    rows_per_w = n_rows // _NW
    n_chunks = rows_per_w // chunk_rows
    n_groups = k_syn // 16
    mesh = plsc.VectorSubcoreMesh(core_axis_name="c", subcore_axis_name="s")

    @functools.partial(
        pl.kernel,
        out_type=jax.ShapeDtypeStruct((n_rows, n_cols), jnp.float32),
        mesh=mesh,
        compiler_params=pltpu.CompilerParams(
            needs_layout_passes=False, use_tc_tiling_on_sc=True),
        scratch_types=[
            pltpu.VMEM((chunk_rows, n_cols), jnp.float32),
            pltpu.VMEM((rows_per_w * k_syn,), jnp.int32),
            pltpu.VMEM((rows_per_w * k_syn,), jnp.float32),
        ],
    )
    def dens(idx_hbm, sign_hbm, w_hbm, buf, idxv, sgnv):
        wid = lax.axis_index("s") * _NC + lax.axis_index("c")
        base_syn = wid * rows_per_w * k_syn
        pltpu.sync_copy(idx_hbm.at[pl.ds(base_syn, rows_per_w * k_syn)], idxv)
        pltpu.sync_copy(sign_hbm.at[pl.ds(base_syn, rows_per_w * k_syn)], sgnv)
        zeros16 = jnp.zeros((16,), jnp.float32)

        def zero_body(i, carry):
            r = i // (n_cols // 16)
            j = i % (n_cols // 16)
            buf[r, pl.ds(j * 16, 16)] = zeros16
            return carry

        lax.fori_loop(0, chunk_rows * n_cols // 16, zero_body, 0)

        def chunk_body(c, carry):
            row0 = c * chunk_rows
            for r in range(chunk_rows):
                rv = jnp.full((16,), r, jnp.int32)
                for g in range(n_groups):
                    s = (row0 + r) * k_syn + g * 16
                    iv = idxv[pl.ds(s, 16)]
                    sv = sgnv[pl.ds(s, 16)]
                    plsc.addupdate_scatter(buf, [rv, iv], sv)
            pltpu.sync_copy(
                buf, w_hbm.at[pl.ds(wid * rows_per_w + row0, chunk_rows)])
            for r in range(chunk_rows):
                rv = jnp.full((16,), r, jnp.int32)
                for g in range(n_groups):
                    s = (row0 + r) * k_syn + g * 16
                    iv = idxv[pl.ds(s, 16)]
                    plsc.store_scatter(buf, [rv, iv], zeros16)
            return carry

        lax.fori_loop(0, n_chunks, chunk_body, 0)

    return dens(idx_flat, sign_flat)


def _tc_encode(x_t_pad, thr_col):
    fp, b = x_t_pad.shape
    ep = fp * NUM_BITS

    def body(x_ref, t_ref, o_ref):
        xp = x_ref[...]
        xe = jnp.broadcast_to(xp[:, None, :], (fp, NUM_BITS, b)).reshape(ep, b)
        o_ref[...] = (xe > t_ref[...]).astype(jnp.bfloat16)

    return pl.pallas_call(
        body,
        out_shape=jax.ShapeDtypeStruct((ep, b), jnp.bfloat16),
    )(x_t_pad, thr_col)


def _tc_layer(w, act, oct_l, theta, block_h):
    h, d = w.shape
    b = act.shape[1]
    c_pad = oct_l.shape[0]

    def body(w_ref, a_ref, oc_ref, o_ref, g_ref, s_ref):
        i = pl.program_id(0)

        @pl.when(i == 0)
        def _():
            g_ref[...] = jnp.zeros_like(g_ref)
            s_ref[...] = jnp.zeros_like(s_ref)

        wb = w_ref[...].astype(jnp.bfloat16)
        z = jnp.dot(wb, a_ref[...], preferred_element_type=jnp.float32)
        a_blk = (z >= theta).astype(jnp.bfloat16)
        o_ref[...] = a_blk
        g_ref[...] += lax.dot_general(
            a_blk, a_blk, (((0,), (0,)), ((), ())),
            preferred_element_type=jnp.float32)
        s_ref[...] += jnp.dot(oc_ref[...], a_blk.astype(jnp.float32),
                              preferred_element_type=jnp.float32)

    return pl.pallas_call(
        body,
        grid=(h // block_h,),
        in_specs=[
            pl.BlockSpec((block_h, d), lambda i: (i, 0)),
            pl.BlockSpec((d, b), lambda i: (0, 0)),
            pl.BlockSpec((c_pad, block_h), lambda i: (0, i)),
        ],
        out_specs=[
            pl.BlockSpec((block_h, b), lambda i: (i, 0)),
            pl.BlockSpec((b, b), lambda i: (0, 0)),
            pl.BlockSpec((c_pad, b), lambda i: (0, 0)),
        ],
        out_shape=[
            jax.ShapeDtypeStruct((h, b), jnp.bfloat16),
            jax.ShapeDtypeStruct((b, b), jnp.float32),
            jax.ShapeDtypeStruct((c_pad, b), jnp.float32),
        ],
    )(w, act, oct_l)


def _tc_final(y_row, g1, g2, soc1, soc2, c_pad):
    b = y_row.shape[1]

    def body(y_ref, g1_ref, g2_ref, s1_ref, s2_ref, o_ref):
        cls = lax.broadcasted_iota(jnp.int32, (c_pad, b), 0)
        yt = (y_ref[...] == cls).astype(jnp.float32)
        g = g1_ref[...] + g2_ref[...]
        o_ref[...] = (s1_ref[...] + s2_ref[...]
                      + jnp.dot(yt, g, preferred_element_type=jnp.float32))

    return pl.pallas_call(
        body,
        out_shape=jax.ShapeDtypeStruct((c_pad, b), jnp.float32),
    )(y_row, g1, g2, soc1, soc2)


def kernel(x, y, idx1, sign1, idx2, sign2, outConn):
    b, f = x.shape
    h, k_syn = idx1.shape
    c = outConn.shape[-1]
    e = f * NUM_BITS

    f_pad = ((f + 7) // 8) * 8
    e_pad = f_pad * NUM_BITS
    c_pad = 16

    x_t = jnp.pad(x.T, ((0, f_pad - f), (0, 0)))
    thr = jnp.linspace(0.0, 1.0, NUM_BITS, dtype=jnp.float32)
    thr_col = jnp.pad(jnp.tile(thr, f), (0, e_pad - e),
                      constant_values=2.0).reshape(e_pad, 1)
    y_row = y.reshape(1, b)
    oc_t = jnp.pad(outConn.transpose(0, 2, 1), ((0, 0), (0, c_pad - c), (0, 0)))

    w1t = _sc_densify(idx1.reshape(-1), sign1.reshape(-1),
                      h, e_pad, k_syn, 16)
    w2t = _sc_densify(idx2.reshape(-1), sign2.reshape(-1),
                      h, h, k_syn, 8)

    enc_t = _tc_encode(x_t, thr_col)
    a1, g1, soc1 = _tc_layer(w1t, enc_t, oc_t[0], THETA1, 256)
    a2, g2, soc2 = _tc_layer(w2t, a1, oc_t[1], THETA2, 256)
    s_t = _tc_final(y_row, g1, g2, soc1, soc2, c_pad)
    return s_t[:c, :].T
